# Initial kernel scaffold; baseline (speedup 1.0000x reference)
import jax
import jax.numpy as jnp
from jax.experimental import pallas as pl


def kernel(w1, b1, w2, b2):
    raise NotImplementedError("write your pallas kernel here")


# speedup vs baseline: 5.3864x; 5.3864x over previous
import functools

import jax
import jax.numpy as jnp
import numpy as np
from jax.experimental import pallas as pl
from jax.experimental.pallas import tpu as pltpu

_RES = [17, 33, 65, 129, 257]
_NPT = [0, 8000, 8000, 100000, 100000]
_RES_LAST = 257


def _interp_mat(old, new):
  g = jnp.linspace(0.0, float(old - 1), new)
  i0 = jnp.floor(g).astype(jnp.int32)
  i1 = jnp.minimum(i0 + 1, old - 1)
  t = (g - i0).astype(jnp.float32)
  m = jnp.zeros((old, new), jnp.float32)
  m = m.at[i0, jnp.arange(new)].add(1.0 - t)
  m = m.at[i1, jnp.arange(new)].add(t)
  return m


def _mlp_body(x_ref, w1_ref, b1_ref, w2_ref, b2_ref, o_ref):
  h = jnp.tanh(
      jnp.dot(x_ref[...], w1_ref[...], preferred_element_type=jnp.float32)
      + b1_ref[...][None, :]
  )
  logit = jnp.sum(h * w2_ref[...][None, :], axis=1) + b2_ref[0]
  o_ref[...] = jax.nn.sigmoid(logit)


def _mlp_query(coords_pad, w1p, b1, w2v, b2, block=2048):
  n_pad = coords_pad.shape[0]
  grid = (n_pad // block,)
  return pl.pallas_call(
      _mlp_body,
      grid=grid,
      in_specs=[
          pl.BlockSpec((block, 8), lambda i: (i, 0)),
          pl.BlockSpec((8, 128), lambda i: (0, 0)),
          pl.BlockSpec((128,), lambda i: (0,)),
          pl.BlockSpec((128,), lambda i: (0,)),
          pl.BlockSpec((1,), lambda i: (0,)),
      ],
      out_specs=pl.BlockSpec((block,), lambda i: (i,)),
      out_shape=jax.ShapeDtypeStruct((n_pad,), jnp.float32),
  )(coords_pad, w1p, b1, w2v, b2)


def _resize_body(a_ref, b_ref, mt_ref, m_ref, o_ref):
  z = pl.program_id(0)
  odd = (z % 2) == 1
  wa = jnp.where(odd, 0.5, 1.0)
  wb = jnp.where(odd, 0.5, 0.0)
  slab = wa * a_ref[0] + wb * b_ref[0]
  t = jnp.dot(mt_ref[...], slab, preferred_element_type=jnp.float32)
  o_ref[0] = jnp.dot(t, m_ref[...], preferred_element_type=jnp.float32)


def _resize(occ, rp, r, m, mt):
  return pl.pallas_call(
      _resize_body,
      grid=(r,),
      in_specs=[
          pl.BlockSpec((1, rp, rp), lambda z: (z // 2, 0, 0)),
          pl.BlockSpec((1, rp, rp), lambda z: (jnp.minimum(z // 2 + 1, rp - 1), 0, 0)),
          pl.BlockSpec((r, rp), lambda z: (0, 0)),
          pl.BlockSpec((rp, r), lambda z: (0, 0)),
      ],
      out_specs=pl.BlockSpec((1, r, r), lambda z: (z, 0, 0)),
      out_shape=jax.ShapeDtypeStruct((r, r, r), jnp.float32),
  )(occ, occ, mt, m)


def _pad_rows(n, mult):
  return (n + mult - 1) // mult * mult


def _coords_from_idx(idx, r):
  wd = r * r
  px = (idx // wd).astype(jnp.float32)
  py = ((idx % wd) // r).astype(jnp.float32)
  pz = (idx % r).astype(jnp.float32)
  stride = float(_RES_LAST - 1) / float(r - 1)
  scale = stride / float(_RES_LAST - 1) * 2.0
  p = jnp.stack([px, py, pz], axis=-1) * scale - 1.0
  return p


def _pad_coords(p):
  k = p.shape[0]
  k_pad = _pad_rows(k, 2048)
  out = jnp.zeros((k_pad, 8), jnp.float32)
  return out.at[:k, :3].set(p)


def kernel(w1, b1, w2, b2):
  w1p = jnp.zeros((8, 128), jnp.float32).at[:3].set(w1)
  w2v = w2[:, 0]

  r0 = _RES[0]
  ar = (jnp.linspace(0.0, float(_RES_LAST - 1), r0).astype(jnp.int32)
        .astype(jnp.float32))
  gi, gj, gk = jnp.meshgrid(ar, ar, ar, indexing='ij')
  grid0 = jnp.stack([gi, gj, gk], axis=-1).reshape(-1, 3)
  grid0 = grid0 / float(_RES_LAST - 1) * 2.0 - 1.0
  n0 = grid0.shape[0]
  occ = _mlp_query(_pad_coords(grid0), w1p, b1, w2v, b2)[:n0]
  occ = occ.reshape(r0, r0, r0)

  for lvl in range(1, len(_RES)):
    rp, r, k = _RES[lvl - 1], _RES[lvl], _NPT[lvl]
    m = _interp_mat(rp, r)
    occ = _resize(occ, rp, r, m, m.T)
    if k <= 0:
      continue
    flat = occ.reshape(-1)
    u = -jnp.abs(flat - 0.5)
    _, idx = jax.lax.top_k(u, k)
    coords = _coords_from_idx(idx, r)
    vals = _mlp_query(_pad_coords(coords), w1p, b1, w2v, b2)[:k]
    occ = flat.at[idx].set(vals).reshape(r, r, r)

  return occ.reshape(1, 1, _RES_LAST, _RES_LAST, _RES_LAST)


# speedup vs baseline: 12.3491x; 2.2927x over previous
import functools

import jax
import jax.numpy as jnp
from jax import lax
from jax.experimental import pallas as pl
from jax.experimental.pallas import tpu as pltpu
from jax.experimental.pallas import tpu_sc as plsc

_RES = [17, 33, 65, 129, 257]
_NPT = [0, 8000, 8000, 100000, 100000]
_RES_LAST = 257

_NW = 32
_HS = 32768
_FL = 2048
_SCPARAMS = pltpu.CompilerParams(needs_layout_passes=False)


def _sc_mesh():
  return plsc.VectorSubcoreMesh(core_axis_name="c", subcore_axis_name="s")


def _interp_mat(old, new):
  g = jnp.linspace(0.0, float(old - 1), new)
  i0 = jnp.floor(g).astype(jnp.int32)
  i1 = jnp.minimum(i0 + 1, old - 1)
  t = (g - i0).astype(jnp.float32)
  m = jnp.zeros((old, new), jnp.float32)
  m = m.at[i0, jnp.arange(new)].add(1.0 - t)
  m = m.at[i1, jnp.arange(new)].add(t)
  return m


def _resize_body(a_ref, b_ref, mt_ref, m_ref, o_ref):
  z = pl.program_id(0)
  odd = (z % 2) == 1
  wa = jnp.where(odd, 0.5, 1.0)
  wb = jnp.where(odd, 0.5, 0.0)
  slab = wa * a_ref[0] + wb * b_ref[0]
  t = jnp.dot(mt_ref[...], slab, preferred_element_type=jnp.float32)
  o_ref[0] = jnp.dot(t, m_ref[...], preferred_element_type=jnp.float32)


def _resize(occ, rp, r, m, mt):
  return pl.pallas_call(
      _resize_body,
      grid=(r,),
      in_specs=[
          pl.BlockSpec((1, rp, rp), lambda z: (z // 2, 0, 0)),
          pl.BlockSpec((1, rp, rp), lambda z: (jnp.minimum(z // 2 + 1, rp - 1), 0, 0)),
          pl.BlockSpec((r, rp), lambda z: (0, 0)),
          pl.BlockSpec((rp, r), lambda z: (0, 0)),
      ],
      out_specs=pl.BlockSpec((1, r, r), lambda z: (z, 0, 0)),
      out_shape=jax.ShapeDtypeStruct((r, r, r), jnp.float32),
      name=f"resize_{r}",
  )(occ, occ, mt, m)


def _mlp_pts_body(sx, px_ref, py_ref, pz_ref, w1x_ref, w1y_ref, w1z_ref,
                  b1_ref, w2_ref, b2_ref, o_ref):
  x = px_ref[...] * sx - 1.0
  y = py_ref[...] * sx - 1.0
  z = pz_ref[...] * sx - 1.0
  a = (x[:, None] * w1x_ref[...][None, :]
       + y[:, None] * w1y_ref[...][None, :]
       + z[:, None] * w1z_ref[...][None, :]
       + b1_ref[...][None, :])
  h = jnp.tanh(a)
  logit = jnp.sum(h * w2_ref[...][None, :], axis=1) + b2_ref[0]
  o_ref[...] = jax.nn.sigmoid(logit)


def _mlp_pts(px, py, pz, sx, w1x, w1y, w1z, b1, w2v, b2, block=2048):
  n = px.shape[0]
  n_pad = (n + block - 1) // block * block
  if n_pad != n:
    pad = jnp.zeros((n_pad - n,), jnp.float32)
    px = jnp.concatenate([px, pad])
    py = jnp.concatenate([py, pad])
    pz = jnp.concatenate([pz, pad])
  vec = lambda i: pl.BlockSpec((block,), lambda i_: (i_,))
  wsp = pl.BlockSpec((128,), lambda i_: (0,))
  out = pl.pallas_call(
      functools.partial(_mlp_pts_body, sx),
      grid=(n_pad // block,),
      in_specs=[vec(0), vec(0), vec(0), wsp, wsp, wsp, wsp, wsp,
                pl.BlockSpec((1,), lambda i_: (0,))],
      out_specs=pl.BlockSpec((block,), lambda i_: (i_,)),
      out_shape=jax.ShapeDtypeStruct((n_pad,), jnp.float32),
      name=f"mlp_{n}",
  )(px, py, pz, w1x, w1y, w1z, b1, w2v, b2)
  return out[:n]


def _zero_vmem(ref, nwords):
  z16 = jnp.zeros((16,), jnp.int32)

  def body(i, _):
    ref[pl.ds(i * 16, 16)] = z16
    return 0

  lax.fori_loop(0, nwords // 16, body, 0)


def _ychunks(r):
  out = []
  y0 = 0
  while y0 < r:
    out.append((y0, min(64, r - y0)))
    y0 += 64
  return out


def _scan_slabs(occ_hbm, dbuf, r, zc, wid, vec_fn):
  nf = r // 16
  n_z = jnp.clip(r - wid * zc, 0, zc)

  def slab_body(i, _):
    z = wid * zc + i
    for (y0, ny) in _ychunks(r):
      pltpu.sync_copy(occ_hbm.at[z, pl.ds(y0, ny)], dbuf.at[pl.ds(0, ny)])

      def row_body(yy, _):
        rowbase = (z * r + y0 + yy) * r
        for j in range(nf):
          v = dbuf[yy, pl.ds(j * 16, 16)]
          d = jnp.abs(v - 0.5)
          key = lax.bitcast_convert_type(d, jnp.int32)
          vec_fn(key, rowbase + j * 16, None)
        v = dbuf[yy, pl.ds(r - 16, 16)]
        d = jnp.abs(v - 0.5)
        key = lax.bitcast_convert_type(d, jnp.int32)
        tmask = lax.iota(jnp.int32, 16) == 15
        vec_fn(key, rowbase + r - 16, tmask)
        return 0

      lax.fori_loop(0, ny, row_body, 0)
    return 0

  lax.fori_loop(0, n_z, slab_body, 0)


def _reduce_hist_to_out(hist, shared, red, out_hbm, c, s):
  pltpu.sync_copy(hist, shared.at[pl.ds(s * _HS, _HS)])
  plsc.subcore_barrier()
  rng = _HS // 16

  def red_body(i, _):
    acc = jnp.zeros((16,), jnp.int32)
    for t in range(16):
      acc = acc + red[t, pl.ds(i * 16, 16)] * 0
    return 0

  for t in range(16):
    pltpu.sync_copy(shared.at[pl.ds(t * _HS + s * rng, rng)], red.at[t])

  def acc_body(i, _):
    acc = red[0, pl.ds(i * 16, 16)]
    for t in range(1, 16):
      acc = acc + red[t, pl.ds(i * 16, 16)]
    hist[pl.ds(s * rng + i * 16, 16)] = acc
    return 0

  lax.fori_loop(0, rng // 16, acc_body, 0)
  pltpu.sync_copy(hist.at[pl.ds(s * rng, rng)],
                  out_hbm.at[pl.ds(c * _HS + s * rng, rng)])


def _crossing_scan(hist_hbm, bufa, bufb, kk):
  state = (jnp.int32(0), jnp.int32(0), jnp.int32(0), jnp.int32(0))
  lanes = lax.iota(jnp.int32, 16)
  for ch in range(_HS // _FL):
    pltpu.sync_copy(hist_hbm.at[pl.ds(ch * _FL, _FL)], bufa)
    pltpu.sync_copy(hist_hbm.at[pl.ds(_HS + ch * _FL, _FL)], bufb)

    def vec_body(i, st):
      vtot, found, b, cb = st
      hv = bufa[pl.ds(i * 16, 16)] + bufb[pl.ds(i * 16, 16)]
      cs = plsc.cumsum(hv)
      cross = (vtot + cs) >= kk
      anyc = plsc.all_reduce_population_count(cross)[0] > 0
      lane = jnp.minimum(plsc.all_reduce_ffs(cross)[0], 15)
      lv = lanes == lane
      hl = jnp.sum(jnp.where(lv, hv, 0))
      cl = jnp.sum(jnp.where(lv, cs, 0))
      newb = ch * _FL + i * 16 + lane
      newcb = vtot + cl - hl
      take = jnp.where(jnp.logical_and(anyc, found == 0), 1, 0)
      b = b * (1 - take) + newb * take
      cb = cb * (1 - take) + newcb * take
      found = jnp.maximum(found, take)
      vtot = vtot + jnp.sum(jnp.where(lanes == 15, cs, 0))
      return (vtot, found, b, cb)

    state = lax.fori_loop(0, _FL // 16, vec_body, state)
  return state[2], state[3]


def _divmod_const(x, d):
  q = jnp.zeros((16,), jnp.int32)
  rem = x
  for b in range(8, -1, -1):
    dv = d << b
    ge = rem >= dv
    rem = jnp.where(ge, rem - dv, rem)
    q = q + jnp.where(ge, 1 << b, 0)
  return q, rem


def _make_k1(r, zc):
  @functools.partial(
      pl.kernel, mesh=_sc_mesh(), compiler_params=_SCPARAMS,
      out_type=jax.ShapeDtypeStruct((2 * _HS,), jnp.int32),
      scratch_types=[
          pltpu.VMEM((_HS,), jnp.int32),
          pltpu.VMEM((64, r), jnp.float32),
          pltpu.VMEM((16, _HS // 16), jnp.int32),
          pltpu.VMEM_SHARED((16 * _HS,), jnp.int32),
      ],
      name=f"sc_hist_hi_{r}",
  )
  def k1(occ_hbm, out_hbm, hist, dbuf, red, shared):
    c = lax.axis_index("c")
    s = lax.axis_index("s")
    wid = c * 16 + s
    _zero_vmem(hist, _HS)
    ones = jnp.full((16,), 1, jnp.int32)

    def vec_fn(key, base, mask):
      b = jnp.minimum(lax.shift_right_logical(key, 15), _HS - 1)
      plsc.addupdate_scatter(hist, [b], ones, mask=mask)

    _scan_slabs(occ_hbm, dbuf, r, zc, wid, vec_fn)
    _reduce_hist_to_out(hist, shared, red, out_hbm, c, s)

  return k1


def _make_k2(r, zc, k):
  @functools.partial(
      pl.kernel, mesh=_sc_mesh(), compiler_params=_SCPARAMS,
      out_type=(jax.ShapeDtypeStruct((2 * _HS,), jnp.int32),
                jax.ShapeDtypeStruct((16,), jnp.int32)),
      scratch_types=[
          pltpu.VMEM((_HS,), jnp.int32),
          pltpu.VMEM((64, r), jnp.float32),
          pltpu.VMEM((16, _HS // 16), jnp.int32),
          pltpu.VMEM((_FL,), jnp.int32),
          pltpu.VMEM((_FL,), jnp.int32),
          pltpu.VMEM_SHARED((16 * _HS,), jnp.int32),
      ],
      name=f"sc_hist_lo_{r}",
  )
  def k2(occ_hbm, h1_hbm, out_hbm, summ_hbm, hist, dbuf, red, bufa, bufb,
         shared):
    c = lax.axis_index("c")
    s = lax.axis_index("s")
    wid = c * 16 + s
    b1, c1 = _crossing_scan(h1_hbm, bufa, bufb, jnp.int32(k))
    _zero_vmem(hist, _HS)
    ones = jnp.full((16,), 1, jnp.int32)

    def vec_fn(key, base, mask):
      hi = lax.shift_right_logical(key, 15)
      lo = jnp.bitwise_and(key, 0x7FFF)
      m = hi == b1
      if mask is not None:
        m = jnp.logical_and(m, mask)
      plsc.addupdate_scatter(hist, [lo], ones, mask=m)

    _scan_slabs(occ_hbm, dbuf, r, zc, wid, vec_fn)
    _reduce_hist_to_out(hist, shared, red, out_hbm, c, s)

    @pl.when(jnp.logical_and(c == 0, s == 0))
    def _():
      lanes = lax.iota(jnp.int32, 16)
      su = jnp.where(lanes == 0, b1, jnp.where(lanes == 1, c1, 0))
      bufa[pl.ds(0, 16)] = su
      pltpu.sync_copy(bufa.at[pl.ds(0, 16)], summ_hbm)

  return k2


def _make_k5(r, zc, k, cap):
  @functools.partial(
      pl.kernel, mesh=_sc_mesh(), compiler_params=_SCPARAMS,
      out_type=(jax.ShapeDtypeStruct((_NW * cap,), jnp.int32),
                jax.ShapeDtypeStruct((_NW * cap,), jnp.int32),
                jax.ShapeDtypeStruct((_NW * 16,), jnp.int32)),
      scratch_types=[
          pltpu.VMEM((64, r), jnp.float32),
          pltpu.VMEM((_FL,), jnp.int32),
          pltpu.VMEM((_FL,), jnp.int32),
          pltpu.VMEM((_FL + 16,), jnp.int32),
          pltpu.VMEM((_FL + 16,), jnp.int32),
          pltpu.SMEM((4,), jnp.int32),
      ],
      name=f"sc_select_{r}",
  )
  def k5(occ_hbm, h2_hbm, summ_hbm, lt_hbm, eq_hbm, cnt_hbm, dbuf, bufa,
         bufb, ltbuf, eqbuf, cnts):
    c = lax.axis_index("c")
    s = lax.axis_index("s")
    wid = c * 16 + s
    pltpu.sync_copy(summ_hbm, bufa.at[pl.ds(0, 16)])
    sv = bufa[pl.ds(0, 16)]
    b1 = sv[0]
    c1 = sv[1]
    b2, c2 = _crossing_scan(h2_hbm, bufa, bufb, jnp.int32(k) - c1)
    thr = jnp.bitwise_or(lax.shift_left(b1, 15), b2)
    cnts[0] = jnp.int32(0)
    cnts[1] = jnp.int32(0)
    cnts[2] = jnp.int32(0)
    cnts[3] = jnp.int32(0)
    lanes = lax.iota(jnp.int32, 16)

    def flush(buf, cnt_i, off_i, spill_hbm):
      @pl.when(cnts[cnt_i] >= _FL)
      def _():
        off = pl.multiple_of(cnts[off_i], _FL)
        pltpu.sync_copy(buf.at[pl.ds(0, _FL)],
                        spill_hbm.at[pl.ds(wid * cap + off, _FL)])
        rv = buf[pl.ds(_FL, 16)]
        buf[pl.ds(0, 16)] = rv
        cnts[cnt_i] = cnts[cnt_i] - _FL
        cnts[off_i] = off + _FL

    def vec_fn(key, base, mask):
      idxv = base + lanes
      m_lt = key < thr
      m_eq = key == thr
      if mask is not None:
        m_lt = jnp.logical_and(m_lt, mask)
        m_eq = jnp.logical_and(m_eq, mask)
      lc = cnts[0]
      plsc.store_compressed(ltbuf.at[pl.ds(lc, 16)], idxv, mask=m_lt)
      cnts[0] = lc + plsc.all_reduce_population_count(m_lt)[0]
      flush(ltbuf, 0, 1, lt_hbm)
      eq_ok = cnts[3] < cap - 4096
      m_eq = jnp.logical_and(m_eq, eq_ok)
      ec = cnts[2]
      plsc.store_compressed(eqbuf.at[pl.ds(ec, 16)], idxv, mask=m_eq)
      cnts[2] = ec + plsc.all_reduce_population_count(m_eq)[0]
      flush(eqbuf, 2, 3, eq_hbm)

    _scan_slabs(occ_hbm, dbuf, r, zc, wid, vec_fn)

    def final_flush(buf, cnt_i, off_i, spill_hbm):
      padded = jnp.bitwise_and(cnts[cnt_i] + 15, ~15)
      loc = jnp.int32(0)
      for bit in (2048, 1024, 512, 256, 128, 64, 32, 16):
        have = jnp.bitwise_and(padded, bit) > 0
        loc_now = loc

        @pl.when(have)
        def _(bit=bit, loc_now=loc_now):
          src_off = pl.multiple_of(loc_now, 16)
          dst_off = pl.multiple_of(cnts[off_i] + loc_now, 16)
          pltpu.sync_copy(
              buf.at[pl.ds(src_off, bit)],
              spill_hbm.at[pl.ds(wid * cap + dst_off, bit)])

        loc = loc + jnp.where(have, bit, 0)

    final_flush(ltbuf, 0, 1, lt_hbm)
    final_flush(eqbuf, 2, 3, eq_hbm)
    tot_lt = cnts[1] + cnts[0]
    tot_eq = cnts[3] + cnts[2]
    cv = jnp.where(lanes == 0, tot_lt, jnp.where(lanes == 1, tot_eq, 0))
    bufa[pl.ds(0, 16)] = cv
    pltpu.sync_copy(bufa.at[pl.ds(0, 16)], cnt_hbm.at[pl.ds(wid * 16, 16)])

  return k5


def _make_k6(r, k, cap):
  wd = r * r

  @functools.partial(
      pl.kernel, mesh=_sc_mesh(), compiler_params=_SCPARAMS,
      out_type=(jax.ShapeDtypeStruct((k,), jnp.int32),
                jax.ShapeDtypeStruct((k,), jnp.float32),
                jax.ShapeDtypeStruct((k,), jnp.float32),
                jax.ShapeDtypeStruct((k,), jnp.float32)),
      scratch_types=[
          pltpu.VMEM((_NW * 16,), jnp.int32),
          pltpu.VMEM((_FL,), jnp.int32),
          pltpu.VMEM((_FL,), jnp.int32),
          pltpu.VMEM((_FL,), jnp.int32),
          pltpu.VMEM((_FL,), jnp.float32),
          pltpu.VMEM((_FL,), jnp.float32),
          pltpu.VMEM((_FL,), jnp.float32),
          pltpu.SemaphoreType.DMA,
      ],
      name=f"sc_place_{r}",
  )
  def k6(cnt_hbm, lt_hbm, eq_hbm, sel_hbm, px_hbm, py_hbm, pz_hbm,
         cvm, lbuf, posb, valb, pxb, pyb, pzb, sem):
    c = lax.axis_index("c")
    s = lax.axis_index("s")
    wid = c * 16 + s
    pltpu.sync_copy(cnt_hbm, cvm)
    lt_pre = jnp.int32(0)
    lt_tot = jnp.int32(0)
    eq_list = []
    lt_list = []
    for v in range(_NW):
      row = cvm[pl.ds(v * 16, 16)]
      lt_v = row[0]
      eq_v = row[1]
      lt_list.append(lt_v)
      eq_list.append(eq_v)
      lt_pre = lt_pre + jnp.where(v < wid, lt_v, 0)
      lt_tot = lt_tot + lt_v
    rquota = jnp.int32(k) - lt_tot
    taken = jnp.int32(0)
    take_pre = jnp.int32(0)
    my_take = jnp.int32(0)
    for v in range(_NW):
      avail = jnp.maximum(rquota - taken, 0)
      take_v = jnp.minimum(eq_list[v], avail)
      take_pre = take_pre + jnp.where(v < wid, take_v, 0)
      my_take = jnp.where(v == wid, take_v, my_take)
      taken = taken + take_v
    my_lt = lt_list[0] * 0
    for v in range(_NW):
      my_lt = jnp.where(v == wid, lt_list[v], my_lt)
    lanes = lax.iota(jnp.int32, 16)

    def move(spill_hbm, n, gbase):
      nb = (n + _FL - 1) // _FL

      def batch(bi, _):
        off = bi * _FL
        pltpu.sync_copy(spill_hbm.at[pl.ds(wid * cap + off, _FL)], lbuf)
        v0 = lbuf[pl.ds(0, 16)][0]
        nrem = n - off

        def vec(i, _):
          li = i * 16 + lanes
          ok = li < nrem
          idxv = jnp.where(ok, lbuf[pl.ds(i * 16, 16)], v0)
          posv = jnp.where(ok, gbase + off + li, gbase + off)
          pxv, rem = _divmod_const(idxv, wd)
          pyv, pzv = _divmod_const(rem, r)
          valb[pl.ds(i * 16, 16)] = idxv
          posb[pl.ds(i * 16, 16)] = posv
          pxb[pl.ds(i * 16, 16)] = pxv.astype(jnp.float32)
          pyb[pl.ds(i * 16, 16)] = pyv.astype(jnp.float32)
          pzb[pl.ds(i * 16, 16)] = pzv.astype(jnp.float32)
          return 0

        lax.fori_loop(0, _FL // 16, vec, 0)
        pltpu.async_copy(valb, sel_hbm.at[posb], sem).wait()
        pltpu.async_copy(pxb, px_hbm.at[posb], sem).wait()
        pltpu.async_copy(pyb, py_hbm.at[posb], sem).wait()
        pltpu.async_copy(pzb, pz_hbm.at[posb], sem).wait()
        return 0

      lax.fori_loop(0, nb, batch, 0)

    move(lt_hbm, my_lt, lt_pre)
    move(eq_hbm, my_take, lt_tot + take_pre)

  return k6


def _make_k7(n_total, c7):
  @functools.partial(
      pl.kernel, mesh=_sc_mesh(), compiler_params=_SCPARAMS,
      out_type=(),
      scratch_types=[
          pltpu.VMEM((c7,), jnp.int32),
          pltpu.VMEM((c7,), jnp.float32),
          pltpu.SemaphoreType.DMA,
      ],
      name=f"sc_scatter_{n_total}",
  )
  def k7(idx_hbm, val_hbm, occ_ref, idxv, valv, sem):
    c = lax.axis_index("c")
    s = lax.axis_index("s")
    wid = c * 16 + s
    pltpu.sync_copy(idx_hbm.at[pl.ds(wid * c7, c7)], idxv)
    pltpu.sync_copy(val_hbm.at[pl.ds(wid * c7, c7)], valv)
    pltpu.async_copy(valv, occ_ref.at[idxv], sem).wait()

  return k7


def _refine_level(occ3d, r, k, w1x, w1y, w1z, b1, w2v, b2):
  zc = (r + _NW - 1) // _NW
  cap = ((k + _FL - 1) // _FL) * _FL + 8192
  h1 = _make_k1(r, zc)(occ3d)
  h2, summ = _make_k2(r, zc, k)(occ3d, h1)
  lt_sp, eq_sp, cnts = _make_k5(r, zc, k, cap)(occ3d, h2, summ)
  sel, px, py, pz = _make_k6(r, k, cap)(cnts, lt_sp, eq_sp)
  stride = (_RES_LAST - 1) // (r - 1)
  sx = 2.0 * float(stride) / float(_RES_LAST - 1)
  vals = _mlp_pts(px, py, pz, sx, w1x, w1y, w1z, b1, w2v, b2)
  c7 = ((k + _NW * 16 - 1) // (_NW * 16)) * 16
  npad = _NW * c7 - k
  selp = jnp.concatenate([sel, jnp.broadcast_to(sel[0:1], (npad,))])
  valp = jnp.concatenate([vals, jnp.broadcast_to(vals[0:1], (npad,))])
  ref = jax.new_ref(occ3d.reshape(-1))
  _make_k7(k, c7)(selp, valp, ref)
  return ref[...].reshape(r, r, r)


def kernel(w1, b1, w2, b2):
  w1x, w1y, w1z = w1[0], w1[1], w1[2]
  w2v = w2[:, 0]

  r0 = _RES[0]
  ar = (jnp.linspace(0.0, float(_RES_LAST - 1), r0).astype(jnp.int32)
        .astype(jnp.float32)) / 16.0
  gi, gj, gk = jnp.meshgrid(ar, ar, ar, indexing="ij")
  occ = _mlp_pts(gi.reshape(-1), gj.reshape(-1), gk.reshape(-1), 0.125,
                 w1x, w1y, w1z, b1, w2v, b2)
  occ = occ.reshape(r0, r0, r0)

  for lvl in range(1, len(_RES)):
    rp, r, k = _RES[lvl - 1], _RES[lvl], _NPT[lvl]
    m = _interp_mat(rp, r)
    occ = _resize(occ, rp, r, m, m.T)
    if k > 0:
      occ = _refine_level(occ, r, k, w1x, w1y, w1z, b1, w2v, b2)

  return occ.reshape(1, 1, _RES_LAST, _RES_LAST, _RES_LAST)


# speedup vs baseline: 47.7297x; 3.8650x over previous
import functools

import jax
import jax.numpy as jnp
from jax import lax
from jax.experimental import pallas as pl
from jax.experimental.pallas import tpu as pltpu
from jax.experimental.pallas import tpu_sc as plsc

_RES = [17, 33, 65, 129, 257]
_NPT = [0, 8000, 8000, 100000, 100000]
_RES_LAST = 257

_NW = 32
_HS = 32768
_FL = 2048
_SCPARAMS = pltpu.CompilerParams(needs_layout_passes=False)


def _sc_mesh():
  return plsc.VectorSubcoreMesh(core_axis_name="c", subcore_axis_name="s")


def _interp_mat(old, new):
  g = jnp.linspace(0.0, float(old - 1), new)
  i0 = jnp.floor(g).astype(jnp.int32)
  i1 = jnp.minimum(i0 + 1, old - 1)
  t = (g - i0).astype(jnp.float32)
  m = jnp.zeros((old, new), jnp.float32)
  m = m.at[i0, jnp.arange(new)].add(1.0 - t)
  m = m.at[i1, jnp.arange(new)].add(t)
  return m


def _resize_body(a_ref, b_ref, mt_ref, m_ref, o_ref):
  z = pl.program_id(0)
  odd = (z % 2) == 1
  wa = jnp.where(odd, 0.5, 1.0)
  wb = jnp.where(odd, 0.5, 0.0)
  slab = wa * a_ref[0] + wb * b_ref[0]
  t = jnp.dot(mt_ref[...], slab, preferred_element_type=jnp.float32)
  o_ref[0] = jnp.dot(t, m_ref[...], preferred_element_type=jnp.float32)


def _resize(occ, rp, r, m, mt):
  return pl.pallas_call(
      _resize_body,
      grid=(r,),
      in_specs=[
          pl.BlockSpec((1, rp, rp), lambda z: (z // 2, 0, 0)),
          pl.BlockSpec((1, rp, rp), lambda z: (jnp.minimum(z // 2 + 1, rp - 1), 0, 0)),
          pl.BlockSpec((r, rp), lambda z: (0, 0)),
          pl.BlockSpec((rp, r), lambda z: (0, 0)),
      ],
      out_specs=pl.BlockSpec((1, r, r), lambda z: (z, 0, 0)),
      out_shape=jax.ShapeDtypeStruct((r, r, r), jnp.float32),
      name=f"resize_{r}",
  )(occ, occ, mt, m)


def _mlp_pts_body(sx, px_ref, py_ref, pz_ref, w1x_ref, w1y_ref, w1z_ref,
                  b1_ref, w2_ref, b2_ref, o_ref):
  x = px_ref[...] * sx - 1.0
  y = py_ref[...] * sx - 1.0
  z = pz_ref[...] * sx - 1.0
  a = (x[:, None] * w1x_ref[...][None, :]
       + y[:, None] * w1y_ref[...][None, :]
       + z[:, None] * w1z_ref[...][None, :]
       + b1_ref[...][None, :])
  h = jnp.tanh(a)
  logit = jnp.sum(h * w2_ref[...][None, :], axis=1) + b2_ref[0]
  o_ref[...] = jax.nn.sigmoid(logit)


def _mlp_pts(px, py, pz, sx, w1x, w1y, w1z, b1, w2v, b2, block=2048):
  n = px.shape[0]
  n_pad = (n + block - 1) // block * block
  if n_pad != n:
    pad = jnp.zeros((n_pad - n,), jnp.float32)
    px = jnp.concatenate([px, pad])
    py = jnp.concatenate([py, pad])
    pz = jnp.concatenate([pz, pad])
  vec = lambda i: pl.BlockSpec((block,), lambda i_: (i_,))
  wsp = pl.BlockSpec((128,), lambda i_: (0,))
  out = pl.pallas_call(
      functools.partial(_mlp_pts_body, sx),
      grid=(n_pad // block,),
      in_specs=[vec(0), vec(0), vec(0), wsp, wsp, wsp, wsp, wsp,
                pl.BlockSpec((1,), lambda i_: (0,))],
      out_specs=pl.BlockSpec((block,), lambda i_: (i_,)),
      out_shape=jax.ShapeDtypeStruct((n_pad,), jnp.float32),
      name=f"mlp_{n}",
  )(px, py, pz, w1x, w1y, w1z, b1, w2v, b2)
  return out[:n]


def _zero_vmem(ref, nwords):
  z16 = jnp.zeros((16,), jnp.int32)

  def body(i, _):
    ref[pl.ds(i * 16, 16)] = z16
    return 0

  lax.fori_loop(0, nwords // 16, body, 0)


def _ychunks(r):
  out = []
  y0 = 0
  while y0 < r:
    out.append((y0, min(64, r - y0)))
    y0 += 64
  return out


def _scan_slabs(occ_hbm, dbuf, r, zc, wid, vec_fn, probe=None):
  nf = r // 16
  n_z = jnp.clip(r - wid * zc, 0, zc)

  def keyof(yy, x0):
    v = dbuf[yy, pl.ds(x0, 16)]
    return lax.bitcast_convert_type(jnp.abs(v - 0.5), jnp.int32)

  def emit(items):
    if probe is None:
      for (key, base, mask) in items:
        vec_fn(key, base, mask)
      return
    m = None
    for (key, base, mask) in items:
      pm = probe(key, mask)
      m = pm if m is None else jnp.logical_or(m, pm)
    anyv = jnp.any(m)

    @pl.when(anyv)
    def _():
      for (key, base, mask) in items:
        vec_fn(key, base, mask)

  def slab_body(i, _):
    z = wid * zc + i
    for (y0, ny) in _ychunks(r):
      pltpu.sync_copy(occ_hbm.at[z, pl.ds(y0, ny)], dbuf.at[pl.ds(0, ny)])

      def row_body(yy, _):
        rowbase = (z * r + y0 + yy) * r
        ngr = nf // 4
        for g in range(ngr):
          emit([(keyof(yy, (g * 4 + u) * 16), rowbase + (g * 4 + u) * 16, None)
                for u in range(4)])
        for j in range(ngr * 4, nf):
          emit([(keyof(yy, j * 16), rowbase + j * 16, None)])
        tmask = lax.iota(jnp.int32, 16) == 15
        emit([(keyof(yy, r - 16), rowbase + r - 16, tmask)])
        return 0

      lax.fori_loop(0, ny, row_body, 0)
    return 0

  lax.fori_loop(0, n_z, slab_body, 0)


def _reduce_hist_to_out(hist, shared, red, out_hbm, c, s):
  pltpu.sync_copy(hist, shared.at[pl.ds(s * _HS, _HS)])
  plsc.subcore_barrier()
  rng = _HS // 16

  def red_body(i, _):
    acc = jnp.zeros((16,), jnp.int32)
    for t in range(16):
      acc = acc + red[t, pl.ds(i * 16, 16)] * 0
    return 0

  for t in range(16):
    pltpu.sync_copy(shared.at[pl.ds(t * _HS + s * rng, rng)], red.at[t])

  def acc_body(i, _):
    acc = red[0, pl.ds(i * 16, 16)]
    for t in range(1, 16):
      acc = acc + red[t, pl.ds(i * 16, 16)]
    hist[pl.ds(s * rng + i * 16, 16)] = acc
    return 0

  lax.fori_loop(0, rng // 16, acc_body, 0)
  pltpu.sync_copy(hist.at[pl.ds(s * rng, rng)],
                  out_hbm.at[pl.ds(c * _HS + s * rng, rng)])


def _crossing_scan(hist_hbm, bufa, bufb, kk):
  state = (jnp.int32(0), jnp.int32(0), jnp.int32(0), jnp.int32(0))
  lanes = lax.iota(jnp.int32, 16)
  for ch in range(_HS // _FL):
    pltpu.sync_copy(hist_hbm.at[pl.ds(ch * _FL, _FL)], bufa)
    pltpu.sync_copy(hist_hbm.at[pl.ds(_HS + ch * _FL, _FL)], bufb)

    def vec_body(i, st):
      vtot, found, b, cb = st
      hv = bufa[pl.ds(i * 16, 16)] + bufb[pl.ds(i * 16, 16)]
      cs = plsc.cumsum(hv)
      cross = (vtot + cs) >= kk
      anyc = plsc.all_reduce_population_count(cross)[0] > 0
      lane = jnp.minimum(plsc.all_reduce_ffs(cross)[0], 15)
      lv = lanes == lane
      hl = jnp.sum(jnp.where(lv, hv, 0))
      cl = jnp.sum(jnp.where(lv, cs, 0))
      newb = ch * _FL + i * 16 + lane
      newcb = vtot + cl - hl
      take = jnp.where(jnp.logical_and(anyc, found == 0), 1, 0)
      b = b * (1 - take) + newb * take
      cb = cb * (1 - take) + newcb * take
      found = jnp.maximum(found, take)
      vtot = vtot + jnp.sum(jnp.where(lanes == 15, cs, 0))
      return (vtot, found, b, cb)

    state = lax.fori_loop(0, _FL // 16, vec_body, state)
  return state[2], state[3]


def _divmod_const(x, d):
  q = jnp.zeros((16,), jnp.int32)
  rem = x
  for b in range(8, -1, -1):
    dv = d << b
    ge = rem >= dv
    rem = jnp.where(ge, rem - dv, rem)
    q = q + jnp.where(ge, 1 << b, 0)
  return q, rem


def _make_k1(r, zc):
  @functools.partial(
      pl.kernel, mesh=_sc_mesh(), compiler_params=_SCPARAMS,
      out_type=jax.ShapeDtypeStruct((2 * _HS,), jnp.int32),
      scratch_types=[
          pltpu.VMEM((_HS,), jnp.int32),
          pltpu.VMEM((64, r), jnp.float32),
          pltpu.VMEM((16, _HS // 16), jnp.int32),
          pltpu.VMEM_SHARED((16 * _HS,), jnp.int32),
      ],
      name=f"sc_hist_hi_{r}",
  )
  def k1(occ_hbm, out_hbm, hist, dbuf, red, shared):
    c = lax.axis_index("c")
    s = lax.axis_index("s")
    wid = c * 16 + s
    _zero_vmem(hist, _HS)
    ones = jnp.full((16,), 1, jnp.int32)

    def vec_fn(key, base, mask):
      b = jnp.minimum(lax.shift_right_logical(key, 15), _HS - 1)
      plsc.addupdate_scatter(hist, [b], ones, mask=mask)

    _scan_slabs(occ_hbm, dbuf, r, zc, wid, vec_fn)
    _reduce_hist_to_out(hist, shared, red, out_hbm, c, s)

  return k1


def _make_k2(r, zc, k):
  @functools.partial(
      pl.kernel, mesh=_sc_mesh(), compiler_params=_SCPARAMS,
      out_type=(jax.ShapeDtypeStruct((2 * _HS,), jnp.int32),
                jax.ShapeDtypeStruct((16,), jnp.int32)),
      scratch_types=[
          pltpu.VMEM((_HS,), jnp.int32),
          pltpu.VMEM((64, r), jnp.float32),
          pltpu.VMEM((16, _HS // 16), jnp.int32),
          pltpu.VMEM((_FL,), jnp.int32),
          pltpu.VMEM((_FL,), jnp.int32),
          pltpu.VMEM_SHARED((16 * _HS,), jnp.int32),
      ],
      name=f"sc_hist_lo_{r}",
  )
  def k2(occ_hbm, h1_hbm, out_hbm, summ_hbm, hist, dbuf, red, bufa, bufb,
         shared):
    c = lax.axis_index("c")
    s = lax.axis_index("s")
    wid = c * 16 + s
    b1, c1 = _crossing_scan(h1_hbm, bufa, bufb, jnp.int32(k))
    _zero_vmem(hist, _HS)
    ones = jnp.full((16,), 1, jnp.int32)

    def vec_fn(key, base, mask):
      hi = lax.shift_right_logical(key, 15)
      lo = jnp.bitwise_and(key, 0x7FFF)
      m = hi == b1
      if mask is not None:
        m = jnp.logical_and(m, mask)
      plsc.addupdate_scatter(hist, [lo], ones, mask=m)

    _scan_slabs(occ_hbm, dbuf, r, zc, wid, vec_fn)
    _reduce_hist_to_out(hist, shared, red, out_hbm, c, s)

    @pl.when(jnp.logical_and(c == 0, s == 0))
    def _():
      lanes = lax.iota(jnp.int32, 16)
      su = jnp.where(lanes == 0, b1, jnp.where(lanes == 1, c1, 0))
      bufa[pl.ds(0, 16)] = su
      pltpu.sync_copy(bufa.at[pl.ds(0, 16)], summ_hbm)

  return k2


def _make_k5(r, zc, k, cap):
  @functools.partial(
      pl.kernel, mesh=_sc_mesh(), compiler_params=_SCPARAMS,
      out_type=(jax.ShapeDtypeStruct((_NW * cap,), jnp.int32),
                jax.ShapeDtypeStruct((_NW * cap,), jnp.int32),
                jax.ShapeDtypeStruct((_NW * 16,), jnp.int32)),
      scratch_types=[
          pltpu.VMEM((64, r), jnp.float32),
          pltpu.VMEM((_FL,), jnp.int32),
          pltpu.VMEM((_FL,), jnp.int32),
          pltpu.VMEM((_FL + 16,), jnp.int32),
          pltpu.VMEM((_FL + 16,), jnp.int32),
          pltpu.SMEM((4,), jnp.int32),
      ],
      name=f"sc_select_{r}",
  )
  def k5(occ_hbm, h2_hbm, summ_hbm, lt_hbm, eq_hbm, cnt_hbm, dbuf, bufa,
         bufb, ltbuf, eqbuf, cnts):
    c = lax.axis_index("c")
    s = lax.axis_index("s")
    wid = c * 16 + s
    pltpu.sync_copy(summ_hbm, bufa.at[pl.ds(0, 16)])
    sv = bufa[pl.ds(0, 16)]
    b1 = sv[0]
    c1 = sv[1]
    b2, c2 = _crossing_scan(h2_hbm, bufa, bufb, jnp.int32(k) - c1)
    thr = jnp.bitwise_or(lax.shift_left(b1, 15), b2)
    cnts[0] = jnp.int32(0)
    cnts[1] = jnp.int32(0)
    cnts[2] = jnp.int32(0)
    cnts[3] = jnp.int32(0)
    lanes = lax.iota(jnp.int32, 16)

    def flush(buf, cnt_i, off_i, spill_hbm):
      @pl.when(cnts[cnt_i] >= _FL)
      def _():
        off = pl.multiple_of(cnts[off_i], _FL)
        pltpu.sync_copy(buf.at[pl.ds(0, _FL)],
                        spill_hbm.at[pl.ds(wid * cap + off, _FL)])
        rv = buf[pl.ds(_FL, 16)]
        buf[pl.ds(0, 16)] = rv
        cnts[cnt_i] = cnts[cnt_i] - _FL
        cnts[off_i] = off + _FL

    def vec_fn(key, base, mask):
      idxv = base + lanes
      m_lt = key < thr
      m_eq = key == thr
      if mask is not None:
        m_lt = jnp.logical_and(m_lt, mask)
        m_eq = jnp.logical_and(m_eq, mask)
      lc = cnts[0]
      plsc.store_compressed(ltbuf.at[pl.ds(lc, 16)], idxv, mask=m_lt)
      cnts[0] = lc + plsc.all_reduce_population_count(m_lt)[0]
      flush(ltbuf, 0, 1, lt_hbm)
      eq_ok = cnts[3] < cap - 4096
      m_eq = jnp.logical_and(m_eq, eq_ok)
      ec = cnts[2]
      plsc.store_compressed(eqbuf.at[pl.ds(ec, 16)], idxv, mask=m_eq)
      cnts[2] = ec + plsc.all_reduce_population_count(m_eq)[0]
      flush(eqbuf, 2, 3, eq_hbm)

    def probe(key, mask):
      m = key <= thr
      if mask is not None:
        m = jnp.logical_and(m, mask)
      return m

    _scan_slabs(occ_hbm, dbuf, r, zc, wid, vec_fn, probe=probe)

    def final_flush(buf, cnt_i, off_i, spill_hbm):
      padded = jnp.bitwise_and(cnts[cnt_i] + 15, ~15)
      loc = jnp.int32(0)
      for bit in (2048, 1024, 512, 256, 128, 64, 32, 16):
        have = jnp.bitwise_and(padded, bit) > 0
        loc_now = loc

        @pl.when(have)
        def _(bit=bit, loc_now=loc_now):
          src_off = pl.multiple_of(loc_now, 16)
          dst_off = pl.multiple_of(cnts[off_i] + loc_now, 16)
          pltpu.sync_copy(
              buf.at[pl.ds(src_off, bit)],
              spill_hbm.at[pl.ds(wid * cap + dst_off, bit)])

        loc = loc + jnp.where(have, bit, 0)

    final_flush(ltbuf, 0, 1, lt_hbm)
    final_flush(eqbuf, 2, 3, eq_hbm)
    tot_lt = cnts[1] + cnts[0]
    tot_eq = cnts[3] + cnts[2]
    cv = jnp.where(lanes == 0, tot_lt, jnp.where(lanes == 1, tot_eq, 0))
    bufa[pl.ds(0, 16)] = cv
    pltpu.sync_copy(bufa.at[pl.ds(0, 16)], cnt_hbm.at[pl.ds(wid * 16, 16)])

  return k5


def _make_k6(r, k, cap, kp):
  wd = r * r

  @functools.partial(
      pl.kernel, mesh=_sc_mesh(), compiler_params=_SCPARAMS,
      out_type=(jax.ShapeDtypeStruct((kp,), jnp.int32),
                jax.ShapeDtypeStruct((kp,), jnp.float32),
                jax.ShapeDtypeStruct((kp,), jnp.float32),
                jax.ShapeDtypeStruct((kp,), jnp.float32)),
      scratch_types=[
          pltpu.VMEM((_NW * 16,), jnp.int32),
          pltpu.VMEM((_FL,), jnp.int32),
          pltpu.VMEM((_FL,), jnp.int32),
          pltpu.VMEM((_FL,), jnp.float32),
          pltpu.VMEM((_FL,), jnp.float32),
          pltpu.VMEM((_FL,), jnp.float32),
      ],
      name=f"sc_place_{r}",
  )
  def k6(cnt_hbm, lt_hbm, eq_hbm, sel_hbm, px_hbm, py_hbm, pz_hbm,
         cvm, lbuf, valb, pxb, pyb, pzb):
    c = lax.axis_index("c")
    s = lax.axis_index("s")
    wid = c * 16 + s
    pltpu.sync_copy(cnt_hbm, cvm)

    def ceil16(x):
      return jnp.bitwise_and(x + 15, -16)

    lt_tot = jnp.int32(0)
    eq_list = []
    lt_list = []
    for v in range(_NW):
      row = cvm[pl.ds(v * 16, 16)]
      lt_list.append(row[0])
      eq_list.append(row[1])
      lt_tot = lt_tot + lt_list[v]
    rquota = jnp.int32(k) - lt_tot
    taken = jnp.int32(0)
    o_w = jnp.int32(0)
    gtot = jnp.int32(0)
    my_take = jnp.int32(0)
    my_lt = jnp.int32(0)
    fw = jnp.int32(_NW)
    for v in range(_NW):
      avail = jnp.maximum(rquota - taken, 0)
      take_v = jnp.minimum(eq_list[v], avail)
      taken = taken + take_v
      p_v = ceil16(lt_list[v]) + ceil16(take_v)
      o_w = o_w + jnp.where(v < wid, p_v, 0)
      gtot = gtot + p_v
      my_take = jnp.where(v == wid, take_v, my_take)
      my_lt = jnp.where(v == wid, lt_list[v], my_lt)
      first_here = jnp.logical_and(fw == _NW, lt_list[v] + take_v > 0)
      fw = jnp.where(first_here, v, fw)
    lanes = lax.iota(jnp.int32, 16)

    def stage_vec(i, idxv):
      pxv, rem = _divmod_const(idxv, wd)
      pyv, pzv = _divmod_const(rem, r)
      valb[pl.ds(i * 16, 16)] = idxv
      pxb[pl.ds(i * 16, 16)] = pxv.astype(jnp.float32)
      pyb[pl.ds(i * 16, 16)] = pyv.astype(jnp.float32)
      pzb[pl.ds(i * 16, 16)] = pzv.astype(jnp.float32)

    def store_all(src_off, dst_off, sz):
      so = pl.multiple_of(src_off, 16)
      do = pl.multiple_of(dst_off, 16)
      pltpu.sync_copy(valb.at[pl.ds(so, sz)], sel_hbm.at[pl.ds(do, sz)])
      pltpu.sync_copy(pxb.at[pl.ds(so, sz)], px_hbm.at[pl.ds(do, sz)])
      pltpu.sync_copy(pyb.at[pl.ds(so, sz)], py_hbm.at[pl.ds(do, sz)])
      pltpu.sync_copy(pzb.at[pl.ds(so, sz)], pz_hbm.at[pl.ds(do, sz)])

    def move(spill_hbm, n, gbase):
      nfull = n // _FL

      def batch(bi, _):
        off = bi * _FL
        pltpu.sync_copy(spill_hbm.at[pl.ds(wid * cap + off, _FL)], lbuf)

        def vec(i, _):
          stage_vec(i, lbuf[pl.ds(i * 16, 16)])
          return 0

        lax.fori_loop(0, _FL // 16, vec, 0)
        store_all(0, gbase + off, _FL)
        return 0

      lax.fori_loop(0, nfull, batch, 0)
      rem_n = n - nfull * _FL

      @pl.when(rem_n > 0)
      def _():
        off = nfull * _FL
        pltpu.sync_copy(spill_hbm.at[pl.ds(wid * cap + off, _FL)], lbuf)
        v0 = lbuf[pl.ds(0, 16)][0]
        nv = (rem_n + 15) // 16

        def vec(i, _):
          ok = (i * 16 + lanes) < rem_n
          stage_vec(i, jnp.where(ok, lbuf[pl.ds(i * 16, 16)], v0))
          return 0

        lax.fori_loop(0, nv, vec, 0)
        padded = nv * 16
        loc = jnp.int32(0)
        for bit in (1024, 512, 256, 128, 64, 32, 16):
          have = jnp.bitwise_and(padded, bit) > 0
          loc_now = loc

          @pl.when(have)
          def _(bit=bit, loc_now=loc_now):
            store_all(loc_now, gbase + off + loc_now, bit)

          loc = loc + jnp.where(have, bit, 0)

    move(lt_hbm, my_lt, o_w)
    move(eq_hbm, my_take, o_w + ceil16(my_lt))

    @pl.when(wid == fw)
    def _():
      @pl.when(my_lt > 0)
      def _():
        pltpu.sync_copy(lt_hbm.at[pl.ds(wid * cap, 16)], lbuf.at[pl.ds(0, 16)])

      @pl.when(my_lt == 0)
      def _():
        pltpu.sync_copy(eq_hbm.at[pl.ds(wid * cap, 16)], lbuf.at[pl.ds(0, 16)])

      v0 = lbuf[pl.ds(0, 16)][0]

      def vec(i, _):
        stage_vec(i, jnp.full((16,), v0, jnp.int32))
        return 0

      lax.fori_loop(0, 64, vec, 0)
      fl = kp - gtot
      loc = jnp.int32(0)
      for bit in (1024, 512, 256, 128, 64, 32, 16):
        have = jnp.bitwise_and(fl, bit) > 0
        loc_now = loc

        @pl.when(have)
        def _(bit=bit, loc_now=loc_now):
          store_all(loc_now, gtot + loc_now, bit)

        loc = loc + jnp.where(have, bit, 0)

  return k6


def _make_k7(n_total, c7):
  @functools.partial(
      pl.kernel, mesh=_sc_mesh(), compiler_params=_SCPARAMS,
      out_type=(),
      scratch_types=[
          pltpu.VMEM((c7,), jnp.int32),
          pltpu.VMEM((c7,), jnp.float32),
          pltpu.SemaphoreType.DMA,
      ],
      name=f"sc_scatter_{n_total}",
  )
  def k7(idx_hbm, val_hbm, occ_ref, idxv, valv, sem):
    c = lax.axis_index("c")
    s = lax.axis_index("s")
    wid = c * 16 + s
    pltpu.sync_copy(idx_hbm.at[pl.ds(wid * c7, c7)], idxv)
    pltpu.sync_copy(val_hbm.at[pl.ds(wid * c7, c7)], valv)
    pltpu.async_copy(valv, occ_ref.at[idxv], sem).wait()

  return k7


def _refine_level(occ3d, r, k, w1x, w1y, w1z, b1, w2v, b2):
  zc = (r + _NW - 1) // _NW
  cap = ((k + _FL - 1) // _FL) * _FL + 8192
  h1 = _make_k1(r, zc)(occ3d)
  h2, summ = _make_k2(r, zc, k)(occ3d, h1)
  lt_sp, eq_sp, cnts = _make_k5(r, zc, k, cap)(occ3d, h2, summ)
  kp = k + 1024
  sel, px, py, pz = _make_k6(r, k, cap, kp)(cnts, lt_sp, eq_sp)
  stride = (_RES_LAST - 1) // (r - 1)
  sx = 2.0 * float(stride) / float(_RES_LAST - 1)
  vals = _mlp_pts(px, py, pz, sx, w1x, w1y, w1z, b1, w2v, b2)
  c7 = ((kp + _NW * 16 - 1) // (_NW * 16)) * 16
  npad = _NW * c7 - kp
  selp = jnp.concatenate([sel, jnp.broadcast_to(sel[0:1], (npad,))])
  valp = jnp.concatenate([vals, jnp.broadcast_to(vals[0:1], (npad,))])
  ref = jax.new_ref(occ3d.reshape(-1))
  _make_k7(kp, c7)(selp, valp, ref)
  return ref[...].reshape(r, r, r)


def kernel(w1, b1, w2, b2):
  w1x, w1y, w1z = w1[0], w1[1], w1[2]
  w2v = w2[:, 0]

  r0 = _RES[0]
  ar = (jnp.linspace(0.0, float(_RES_LAST - 1), r0).astype(jnp.int32)
        .astype(jnp.float32)) / 16.0
  gi, gj, gk = jnp.meshgrid(ar, ar, ar, indexing="ij")
  occ = _mlp_pts(gi.reshape(-1), gj.reshape(-1), gk.reshape(-1), 0.125,
                 w1x, w1y, w1z, b1, w2v, b2)
  occ = occ.reshape(r0, r0, r0)

  for lvl in range(1, len(_RES)):
    rp, r, k = _RES[lvl - 1], _RES[lvl], _NPT[lvl]
    m = _interp_mat(rp, r)
    occ = _resize(occ, rp, r, m, m.T)
    if k > 0:
      occ = _refine_level(occ, r, k, w1x, w1y, w1z, b1, w2v, b2)

  return occ.reshape(1, 1, _RES_LAST, _RES_LAST, _RES_LAST)
